# Initial kernel scaffold; baseline (speedup 1.0000x reference)
import jax
import jax.numpy as jnp
from jax.experimental import pallas as pl


def kernel(x, mean, variance, prior):
    raise NotImplementedError("write your pallas kernel here")


# speedup vs baseline: 2.0145x; 2.0145x over previous
import math

import jax
import jax.numpy as jnp
from jax.experimental import pallas as pl
from jax.experimental.pallas import tpu as pltpu

_EPS = 1e-3
_CHUNK = 256


def _ucb_kernel(x_ref, mean_ref, var_ref, out_ref, p_scr, r_scr, s_acc):
    ph = pl.program_id(0)
    b = pl.program_id(1)
    n_t, n_d = x_ref.shape[1], x_ref.shape[2]
    n_k = mean_ref.shape[0]
    n_ch = n_t // _CHUNK

    mean = mean_ref[...]
    nc = (-0.5) / (jax.nn.softplus(var_ref[...]) + _EPS)

    @pl.when(ph == 0)
    def _pass0():
        @pl.when(b == 0)
        def _init():
            s_acc[...] = jnp.zeros_like(s_acc)

        acc = [jnp.zeros((1, n_d), jnp.float32) for _ in range(n_k)]
        for c in range(n_ch):
            sl = slice(c * _CHUNK, (c + 1) * _CHUNK)
            xc = x_ref[0, sl, :]
            s = None
            for k in range(n_k):
                d = xc - mean[k : k + 1, :]
                e = jnp.exp(d * d * nc[k : k + 1, :])
                p_scr[k, sl, :] = e
                s = e if s is None else s + e
            r = 1.0 / (s + _EPS)
            for k in range(n_k):
                acc[k] = acc[k] + jnp.sum(
                    p_scr[k, sl, :] * r, axis=0, keepdims=True
                )
        s_acc[...] = s_acc[...] + jnp.concatenate(acc, axis=0)

    @pl.when(ph == 1)
    def _pass1():
        r_s = 1.0 / (s_acc[...] + _EPS)
        a1 = [jnp.zeros((1, n_d), jnp.float32) for _ in range(n_k)]
        a3 = [jnp.zeros((1, n_d), jnp.float32) for _ in range(n_k)]
        for c in range(n_ch):
            sl = slice(c * _CHUNK, (c + 1) * _CHUNK)
            xc = x_ref[0, sl, :]
            s = None
            for k in range(n_k):
                d = xc - mean[k : k + 1, :]
                e = jnp.exp(d * d * nc[k : k + 1, :])
                p_scr[k, sl, :] = e
                s = e if s is None else s + e
            r = 1.0 / (s + _EPS)
            r_scr[sl, :] = r
            xr = xc * r
            for k in range(n_k):
                e = p_scr[k, sl, :]
                tau = e * r
                t1 = e * xr
                t3 = t1 * t1 * tau
                a1[k] = a1[k] + jnp.sum(t1, axis=0, keepdims=True)
                a3[k] = a3[k] + jnp.sum(t3, axis=0, keepdims=True)

        pri_scale = jnp.float32(1.0 / math.sqrt(1.0 + _EPS))
        inv_t = jnp.float32(1.0 / n_t)
        w = []
        cc = []
        for k in range(n_k):
            rs_k = r_s[k : k + 1, :]
            e1 = rs_k * a1[k] * inv_t
            e3 = (rs_k * rs_k * rs_k) * a3[k] * inv_t
            wk = jax.lax.rsqrt(e3 + _EPS) * pri_scale
            w.append(wk)
            cc.append(e1 * wk)

        for c in range(n_ch):
            sl = slice(c * _CHUNK, (c + 1) * _CHUNK)
            xc = x_ref[0, sl, :]
            r = r_scr[sl, :]
            p_sum = None
            q_sum = None
            for k in range(n_k):
                e = p_scr[k, sl, :]
                p_sum = e * w[k] if p_sum is None else p_sum + e * w[k]
                q_sum = e * cc[k] if q_sum is None else q_sum + e * cc[k]
            out_ref[0, sl, :] = r * (xc * p_sum - q_sum)


def kernel(x, mean, variance, prior):
    del prior
    n_b, n_t, n_d = x.shape
    n_k = mean.shape[0]
    return pl.pallas_call(
        _ucb_kernel,
        grid=(2, n_b),
        in_specs=[
            pl.BlockSpec((1, n_t, n_d), lambda ph, b: (b, 0, 0)),
            pl.BlockSpec((n_k, n_d), lambda ph, b: (0, 0)),
            pl.BlockSpec((n_k, n_d), lambda ph, b: (0, 0)),
        ],
        out_specs=pl.BlockSpec((1, n_t, n_d), lambda ph, b: (b * ph, 0, 0)),
        out_shape=jax.ShapeDtypeStruct((n_b, n_t, n_d), jnp.float32),
        scratch_shapes=[
            pltpu.VMEM((n_k, n_t, n_d), jnp.float32),
            pltpu.VMEM((n_t, n_d), jnp.float32),
            pltpu.VMEM((n_k, n_d), jnp.float32),
        ],
        compiler_params=pltpu.CompilerParams(
            dimension_semantics=("arbitrary", "arbitrary"),
        ),
        name="ucb_norm",
    )(x, mean, variance)


# speedup vs baseline: 2.4013x; 1.1920x over previous
import math

import jax
import jax.numpy as jnp
from jax.experimental import pallas as pl
from jax.experimental.pallas import tpu as pltpu

_EPS = 1e-3
_CHUNK = 128
_LOG2E = math.log2(math.e)


def _ucb_kernel(x_ref, mean_ref, var_ref, out_ref, s_acc, a1_scr, a3_scr):
    ph = pl.program_id(0)
    b = pl.program_id(1)
    n_t, n_d = x_ref.shape[1], x_ref.shape[2]
    n_k = mean_ref.shape[0]
    n_ch = n_t // _CHUNK

    mean = mean_ref[...]
    nc2 = (-0.5 * _LOG2E) / (jax.nn.softplus(var_ref[...]) + _EPS)

    def _vsum(v):
        return jnp.sum(v.reshape(_CHUNK // 8, 8, n_d), axis=0)

    @pl.when(ph == 0)
    def _stats_pass():
        @pl.when(b == 0)
        def _init():
            s_acc[...] = jnp.zeros_like(s_acc)

        s8 = [jnp.zeros((8, n_d), jnp.float32) for _ in range(n_k)]
        a1 = [jnp.zeros((8, n_d), jnp.float32) for _ in range(n_k)]
        a3 = [jnp.zeros((8, n_d), jnp.float32) for _ in range(n_k)]
        for c in range(n_ch):
            sl = slice(c * _CHUNK, (c + 1) * _CHUNK)
            xc = x_ref[0, sl, :]
            es = []
            s = None
            for k in range(n_k):
                d = xc - mean[k : k + 1, :]
                e = jnp.exp2(d * d * nc2[k : k + 1, :])
                es.append(e)
                s = e if s is None else s + e
            r = 1.0 / (s + _EPS)
            for k in range(n_k):
                tau = es[k] * r
                t1 = tau * xc
                t3 = t1 * t1 * tau
                s8[k] = s8[k] + _vsum(tau)
                a1[k] = a1[k] + _vsum(t1)
                a3[k] = a3[k] + _vsum(t3)
        fin = lambda vs: jnp.concatenate(
            [jnp.sum(v, axis=0, keepdims=True) for v in vs], axis=0
        )
        s_acc[...] = s_acc[...] + fin(s8)
        a1_scr[b] = fin(a1)
        a3_scr[b] = fin(a3)

    @pl.when(ph == 1)
    def _output_pass():
        r_s = 1.0 / (s_acc[...] + _EPS)
        pri_scale = jnp.float32(1.0 / math.sqrt(1.0 + _EPS))
        inv_t = jnp.float32(1.0 / n_t)
        a1 = a1_scr[b]
        a3 = a3_scr[b]
        e1 = r_s * a1 * inv_t
        e3 = (r_s * r_s * r_s) * a3 * inv_t
        w = jax.lax.rsqrt(e3 + _EPS) * pri_scale
        cw = e1 * w

        for c in range(n_ch):
            sl = slice(c * _CHUNK, (c + 1) * _CHUNK)
            xc = x_ref[0, sl, :]
            es = []
            s = None
            for k in range(n_k):
                d = xc - mean[k : k + 1, :]
                e = jnp.exp2(d * d * nc2[k : k + 1, :])
                es.append(e)
                s = e if s is None else s + e
            r = 1.0 / (s + _EPS)
            p_sum = None
            q_sum = None
            for k in range(n_k):
                pk = es[k] * w[k : k + 1, :]
                qk = es[k] * cw[k : k + 1, :]
                p_sum = pk if p_sum is None else p_sum + pk
                q_sum = qk if q_sum is None else q_sum + qk
            out_ref[0, sl, :] = r * (xc * p_sum - q_sum)


def kernel(x, mean, variance, prior):
    del prior
    n_b, n_t, n_d = x.shape
    n_k = mean.shape[0]
    return pl.pallas_call(
        _ucb_kernel,
        grid=(2, n_b),
        in_specs=[
            pl.BlockSpec((1, n_t, n_d), lambda ph, b: (b, 0, 0)),
            pl.BlockSpec((n_k, n_d), lambda ph, b: (0, 0)),
            pl.BlockSpec((n_k, n_d), lambda ph, b: (0, 0)),
        ],
        out_specs=pl.BlockSpec((1, n_t, n_d), lambda ph, b: (b * ph, 0, 0)),
        out_shape=jax.ShapeDtypeStruct((n_b, n_t, n_d), jnp.float32),
        scratch_shapes=[
            pltpu.VMEM((n_k, n_d), jnp.float32),
            pltpu.VMEM((n_b, n_k, n_d), jnp.float32),
            pltpu.VMEM((n_b, n_k, n_d), jnp.float32),
        ],
        compiler_params=pltpu.CompilerParams(
            dimension_semantics=("arbitrary", "arbitrary"),
        ),
        name="ucb_norm",
    )(x, mean, variance)
